# baseline (speedup 1.0000x reference)
import jax
import jax.numpy as jnp
from jax.experimental import pallas as pl
from jax.experimental.pallas import tpu as pltpu

_IN_P = 8
_HID_P = 128
_W2_OFF = _IN_P
_W3_OFF = _IN_P + _HID_P
_B_OFF = _IN_P + 2 * _HID_P

_HID = 32
_OUT_W = 8
_Y_W = 16

_T_W1 = 0
_T_W2 = 32
_T_W3 = 64
_T_ROWS = 80
_B_ROWS = 32


def _round_up(x, m):
    return ((x + m - 1) // m) * m


def _prep_params(packed):
    w1 = packed[0:_IN_P, 0:_HID]
    w2 = packed[_W2_OFF:_W2_OFF + _HID, 0:_HID]
    w3 = packed[_W3_OFF:_W3_OFF + _HID, 0:_OUT_W]
    wt = jnp.zeros((_T_ROWS, _HID_P), jnp.bfloat16)
    wt = wt.at[_T_W1:_T_W1 + _HID, 0:_IN_P].set(w1.T.astype(jnp.bfloat16))
    wt = wt.at[_T_W2:_T_W2 + _HID, 0:_HID].set(w2.T.astype(jnp.bfloat16))
    wt = wt.at[_T_W3:_T_W3 + _OUT_W, 0:_HID].set(w3.T.astype(jnp.bfloat16))
    bs = jnp.zeros((_B_ROWS, _HID_P), jnp.float32)
    bs = bs.at[0:_HID, 0].set(packed[_B_OFF + 0, 0:_HID])
    bs = bs.at[0:_HID, 1].set(packed[_B_OFF + 1, 0:_HID])
    bs = bs.at[0:_OUT_W, 2].set(packed[_B_OFF + 2, 0:_OUT_W])
    return wt, bs


def _mlp_t_body(x_ref, w_ref, b_ref, o_ref):
    xt = x_ref[...]

    w1t = w_ref[_T_W1:_T_W1 + _HID, 0:_IN_P]
    w2t = w_ref[_T_W2:_T_W2 + _HID, 0:_HID]
    w3t = w_ref[_T_W3:_T_W3 + _Y_W, 0:_HID]
    b1c = b_ref[0:_HID, 0:1].astype(jnp.bfloat16)
    b2c = b_ref[0:_HID, 1:2].astype(jnp.bfloat16)
    b3c = b_ref[0:_OUT_W, 2:3]

    h = jnp.dot(w1t, xt, preferred_element_type=jnp.float32)
    h = jnp.maximum(h.astype(jnp.bfloat16) + b1c, 0)
    h = jnp.dot(w2t, h, preferred_element_type=jnp.float32)
    h = jnp.maximum(h.astype(jnp.bfloat16) + b2c, 0)
    y = jnp.dot(w3t, h, preferred_element_type=jnp.float32)
    y = y[0:_OUT_W, :] + b3c
    o_ref[...] = (jnp.tanh(y) * 10.0).astype(o_ref.dtype)


def kernel(x, packed_params, *, tile_b=131072):
    B, in_dim = x.shape

    tb = min(tile_b, _round_up(max(B, 1), 128))
    Bp = _round_up(B, tb)

    xt = jnp.zeros((_IN_P, Bp), jnp.bfloat16)
    xt = xt.at[:in_dim, :B].set(x.T.astype(jnp.bfloat16))
    wt, bs = _prep_params(packed_params)

    out = pl.pallas_call(
        _mlp_t_body,
        out_shape=jax.ShapeDtypeStruct((_OUT_W, Bp), jnp.bfloat16),
        grid=(Bp // tb,),
        in_specs=[
            pl.BlockSpec((_IN_P, tb), lambda i: (0, i)),
            pl.BlockSpec((_T_ROWS, _HID_P), lambda i: (0, 0)),
            pl.BlockSpec((_B_ROWS, _HID_P), lambda i: (0, 0)),
        ],
        out_specs=pl.BlockSpec((_OUT_W, tb), lambda i: (0, i)),
        compiler_params=pltpu.CompilerParams(
            dimension_semantics=("parallel",)),
    )(xt, wt, bs)

    return out[:2, :B].T.astype(jnp.float32)


# speedup vs baseline: 1.0084x; 1.0084x over previous
import jax
import jax.numpy as jnp
from jax.experimental import pallas as pl
from jax.experimental.pallas import tpu as pltpu

_IN_P = 8
_HID_P = 128
_W2_OFF = _IN_P
_W3_OFF = _IN_P + _HID_P
_B_OFF = _IN_P + 2 * _HID_P

_HID = 32
_OUT_W = 8
_Y_W = 16

_T_W1 = 0
_T_W2 = 32
_T_W3 = 64
_T_ROWS = 80
_B_ROWS = 32


def _round_up(x, m):
    return ((x + m - 1) // m) * m


def _prep_params(packed):
    w1 = packed[0:_IN_P, 0:_HID]
    w2 = packed[_W2_OFF:_W2_OFF + _HID, 0:_HID]
    w3 = packed[_W3_OFF:_W3_OFF + _HID, 0:_OUT_W]
    wt = jnp.zeros((_T_ROWS, _HID_P), jnp.bfloat16)
    wt = wt.at[_T_W1:_T_W1 + _HID, 0:_IN_P].set(w1.T.astype(jnp.bfloat16))
    wt = wt.at[_T_W2:_T_W2 + _HID, 0:_HID].set(w2.T.astype(jnp.bfloat16))
    wt = wt.at[_T_W3:_T_W3 + _OUT_W, 0:_HID].set(w3.T.astype(jnp.bfloat16))
    bs = jnp.zeros((_B_ROWS, _HID_P), jnp.float32)
    bs = bs.at[0:_HID, 0].set(packed[_B_OFF + 0, 0:_HID])
    bs = bs.at[0:_HID, 1].set(packed[_B_OFF + 1, 0:_HID])
    bs = bs.at[0:_OUT_W, 2].set(packed[_B_OFF + 2, 0:_OUT_W])
    return wt, bs


def _mlp_t_body(x_ref, w_ref, b_ref, o_ref):
    xt = x_ref[...]

    w1t = w_ref[_T_W1:_T_W1 + _HID, 0:_IN_P]
    w2t = w_ref[_T_W2:_T_W2 + _HID, 0:_HID]
    w3t = w_ref[_T_W3:_T_W3 + _Y_W, 0:_HID]
    b1c = b_ref[0:_HID, 0:1]
    b2c = b_ref[0:_HID, 1:2]
    b3c = b_ref[0:_OUT_W, 2:3]

    h = jnp.dot(w1t, xt, preferred_element_type=jnp.float32) + b1c
    h = jnp.maximum(h.astype(jnp.bfloat16), 0)
    h = jnp.dot(w2t, h, preferred_element_type=jnp.float32) + b2c
    h = jnp.maximum(h.astype(jnp.bfloat16), 0)
    y = jnp.dot(w3t, h, preferred_element_type=jnp.float32)
    y = y[0:_OUT_W, :] + b3c
    o_ref[...] = (jnp.tanh(y) * 10.0).astype(o_ref.dtype)


def kernel(x, packed_params, *, tile_b=131072):
    B, in_dim = x.shape

    tb = min(tile_b, _round_up(max(B, 1), 128))
    Bp = _round_up(B, tb)

    xt = jnp.zeros((_IN_P, Bp), jnp.bfloat16)
    xt = xt.at[:in_dim, :B].set(x.T.astype(jnp.bfloat16))
    wt, bs = _prep_params(packed_params)

    out = pl.pallas_call(
        _mlp_t_body,
        out_shape=jax.ShapeDtypeStruct((_OUT_W, Bp), jnp.bfloat16),
        grid=(Bp // tb,),
        in_specs=[
            pl.BlockSpec((_IN_P, tb), lambda i: (0, i)),
            pl.BlockSpec((_T_ROWS, _HID_P), lambda i: (0, 0)),
            pl.BlockSpec((_B_ROWS, _HID_P), lambda i: (0, 0)),
        ],
        out_specs=pl.BlockSpec((_OUT_W, tb), lambda i: (0, i)),
        compiler_params=pltpu.CompilerParams(
            dimension_semantics=("parallel",)),
    )(xt, wt, bs)

    return out[:2, :B].T.astype(jnp.float32)
